# baseline (speedup 1.0000x reference)
import functools

import jax
import jax.numpy as jnp
from jax import lax
from jax.experimental import pallas as pl
from jax.experimental.pallas import tpu as pltpu
from jax.experimental.pallas import tpu_sc as plsc

_NC = 2
_NS = 16
_NW = _NC * _NS
_DEPTH = 4
_FLUSH = 64


def _sc_gather_sweep(uidx, iidx, utab3, itab3):
    batch = uidx.shape[0]
    n_tr, n_sub, n_ids = utab3.shape
    hidden = n_tr * n_sub
    n_blk = (n_ids + 127) // 128
    n_vec = batch // 16
    out_rows = batch + 16
    max_my = (n_blk + _NW - 1) // _NW + 1
    max_chunks = (max_my + 15) // 16
    mesh = plsc.VectorSubcoreMesh(core_axis_name="c", subcore_axis_name="s")

    block = pltpu.VMEM((n_tr, n_sub, 128), jnp.float32)

    @functools.partial(
        pl.kernel,
        mesh=mesh,
        compiler_params=pltpu.CompilerParams(needs_layout_passes=False),
        out_type=(
            jax.ShapeDtypeStruct((out_rows, 2 * hidden), jnp.float32),
            jax.ShapeDtypeStruct((out_rows, 2 * hidden), jnp.float32),
        ),
        scratch_types=(
            pltpu.VMEM((2048,), jnp.int32),
            pltpu.VMEM((272,), jnp.int32),
            pltpu.VMEM((272,), jnp.int32),
            pltpu.VMEM((272,), jnp.int32),
            pltpu.VMEM((batch + 16,), jnp.int32),
            [block] * _DEPTH,
            pltpu.VMEM((_FLUSH, 2 * hidden), jnp.float32),
            pltpu.VMEM((_FLUSH,), jnp.int32),
            [pltpu.SemaphoreType.DMA] * _DEPTH,
        ),
    )
    def gather_kernel(uidx_hbm, iidx_hbm, utab_hbm, itab_hbm,
                      uout_hbm, iout_hbm,
                      idbuf, hist, cum, ptr, hits, blks, stage, slots, sems):
        wid = lax.axis_index("s") * _NC + lax.axis_index("c")
        lo = (wid * n_blk) // _NW
        hi = ((wid + 1) * n_blk) // _NW
        lanes = lax.iota(jnp.int32, 16)
        zeros16 = lanes * 0
        trq = [(lanes + 16 * q) >> 3 for q in range(hidden // 16)]
        sq = [(lanes + 16 * q) & 7 for q in range(hidden // 16)]

        def clear272(ref):
            for v in range(272 // 16):
                ref[pl.ds(16 * v, 16)] = zeros16

        def build_hits(idx_hbm):
            clear272(hist)

            def hist_chunk(c, carry):
                pltpu.sync_copy(idx_hbm.at[pl.ds(c * 2048, 2048)], idbuf)

                def hist_vec(v, carry2):
                    ids = idbuf[pl.ds(16 * v, 16)]
                    blk = lax.shift_right_logical(ids, 7)
                    mine = (blk >= lo) & (blk < hi)
                    bl = jnp.where(mine, blk - lo, 0)
                    plsc.addupdate_scatter(
                        hist, [bl], zeros16 + 1, mask=mine)
                    return carry2

                lax.fori_loop(0, 2048 // 16, hist_vec, 0)
                return carry

            lax.fori_loop(0, batch // 2048, hist_chunk, 0)
            running = jnp.int32(0)
            for v in range(272 // 16):
                h = hist[pl.ds(16 * v, 16)]
                incl = plsc.cumsum(h)
                excl = incl - h + running
                cum[pl.ds(16 * v, 16)] = excl
                ptr[pl.ds(16 * v, 16)] = excl
                running = running + incl[15]
            def sort_chunk(c, carry):
                pltpu.sync_copy(idx_hbm.at[pl.ds(c * 2048, 2048)], idbuf)

                def sort_vec(v, carry2):
                    ids = idbuf[pl.ds(16 * v, 16)]
                    blk = lax.shift_right_logical(ids, 7)
                    mine = (blk >= lo) & (blk < hi)
                    rank, _ = plsc.scan_count(blk, mine)
                    bl = jnp.where(mine, blk - lo, 0)
                    pos = plsc.load_gather(ptr, [bl]) + rank - 1
                    slot = (c * 2048 + v * 16) + lanes
                    rec = slot * 128 + (ids & 127)
                    pos = jnp.where(mine, pos, jnp.int32(batch))
                    plsc.store_scatter(hits, [pos], rec, mask=mine)
                    plsc.addupdate_scatter(
                        ptr, [bl], zeros16 + 1, mask=mine)
                    return carry2

                lax.fori_loop(0, 2048 // 16, sort_vec, 0)
                return carry

            lax.fori_loop(0, batch // 2048, sort_chunk, 0)

        def fire(j, d, tab_hbm):
            off = pl.multiple_of((lo + j) * 128, 128)
            pltpu.async_copy(tab_hbm.at[:, :, pl.ds(off, 128)],
                             blks[d], sems[d])

        def wait_blk(d, tab_hbm):
            pltpu.make_async_copy(tab_hbm.at[:, :, pl.ds(0, 128)],
                                  blks[d], sems[d]).wait()

        def sweep(tab_hbm, out_hbm, col0, hc0):
            n_my = hi - lo

            for d in range(_DEPTH):
                @pl.when(d < n_my)
                def _():
                    fire(d, d, tab_hbm)

            def do_chunk(k, c0, c1, d, hc):
                h0 = c0 + 16 * k
                rec = hits[pl.ds(h0, 16)]
                valid = (h0 + lanes) < c1
                slot = jnp.where(valid,
                                 lax.shift_right_logical(rec, 7),
                                 jnp.int32(batch))
                lane_v = rec & 127
                r0 = hc & (_FLUSH - 1)
                slots[pl.ds(r0, 16)] = slot
                for l in range(16):
                    lv = zeros16 + lane_v[l]
                    r = r0 + l
                    for q in range(hidden // 16):
                        val = plsc.load_gather(blks[d], [trq[q], sq[q], lv])
                        stage[r, pl.ds(col0 + 16 * q, 16)] = val
                new_hc = hc + 16

                @pl.when((new_hc & (_FLUSH - 1)) == 0)
                def _():
                    pltpu.sync_copy(stage, out_hbm.at[slots])

                return new_hc

            def chunk_body(jc, hc):
                cv0 = cum[pl.ds(jc * 16, 16)]
                cv1 = cum[pl.ds(jc * 16 + 16, 16)]
                for l in range(16):
                    d = l % _DEPTH
                    j = jc * 16 + l

                    @pl.when(j < n_my)
                    def _():
                        wait_blk(d, tab_hbm)
                    c0 = cv0[l]
                    c1 = cv1[0] if l == 15 else cv0[l + 1]
                    n_ch = lax.shift_right_logical(c1 - c0 + 15, 4)

                    def ch_body(k, hc_in):
                        return do_chunk(k, c0, c1, d, hc_in)

                    hc = lax.fori_loop(0, jnp.where(j < n_my, n_ch, 0),
                                       ch_body, hc)

                    @pl.when(j + _DEPTH < n_my)
                    def _():
                        fire(j + _DEPTH, d, tab_hbm)
                return hc

            hc = lax.fori_loop(0, max_chunks, chunk_body, hc0)
            pltpu.sync_copy(stage, out_hbm.at[slots])
            return hc

        def zero_stage(col0, width):
            def zrow(r, carry):
                def zcol(q, carry2):
                    stage[r, pl.ds(col0 + 16 * q, 16)] = (
                        zeros16.astype(jnp.float32))
                    return carry2

                lax.fori_loop(0, width // 16, zcol, 0)
                return carry

            lax.fori_loop(0, _FLUSH, zrow, 0)

            def zslot(v, carry):
                slots[pl.ds(16 * v, 16)] = zeros16 + batch
                return carry

            lax.fori_loop(0, _FLUSH // 16, zslot, 0)

        zero_stage(0, 2 * hidden)
        build_hits(uidx_hbm)
        sweep(utab_hbm, uout_hbm, 0, jnp.int32(0))

        zero_stage(0, hidden)

        build_hits(iidx_hbm)
        sweep(itab_hbm, iout_hbm, hidden, jnp.int32(0))

    return gather_kernel(uidx, iidx, utab3, itab3)


def _mlp_body(xu_ref, xi_ref, w1_ref, b1_ref, w2_ref, b2_ref, w3_ref, b3_ref,
              o_ref):
    x = xu_ref[...] + xi_ref[...]
    h = jnp.dot(x, w1_ref[...], preferred_element_type=jnp.float32)
    h = jnp.maximum(h + b1_ref[...], 0.0)
    h = jnp.maximum(
        jnp.dot(h, w2_ref[...], preferred_element_type=jnp.float32)
        + b2_ref[...], 0.0)
    o_ref[...] = (jnp.dot(h, w3_ref[...], preferred_element_type=jnp.float32)
                  + b3_ref[...])


def _tc_mlp(xu, xi, w1, b1r, w2, b2r, w3, b3r, batch, blk):
    d_in = xu.shape[1]
    d1 = w1.shape[1]
    d2 = w2.shape[1]
    d3 = w3.shape[1]
    grid = (batch // blk,)
    fixed = lambda b: (0, 0)
    return pl.pallas_call(
        _mlp_body,
        grid=grid,
        in_specs=[
            pl.BlockSpec((blk, d_in), lambda b: (b, 0)),
            pl.BlockSpec((blk, d_in), lambda b: (b, 0)),
            pl.BlockSpec((d_in, d1), fixed),
            pl.BlockSpec((1, d1), fixed),
            pl.BlockSpec((d1, d2), fixed),
            pl.BlockSpec((1, d2), fixed),
            pl.BlockSpec((d2, d3), fixed),
            pl.BlockSpec((1, d3), fixed),
        ],
        out_specs=pl.BlockSpec((blk, d3), lambda b: (b, 0)),
        out_shape=jax.ShapeDtypeStruct((batch, d3), jnp.float32),
    )(xu, xi, w1, b1r, w2, b2r, w3, b3r)


def kernel(user_id, item_id, user_table, item_table, W1, b1, W2, b2, W3, b3):
    n_ids, hidden = user_table.shape
    batch = user_id.shape[0]
    utab3 = user_table.T.reshape(8, hidden // 8, n_ids)
    itab3 = item_table.T.reshape(8, hidden // 8, n_ids)
    xu, xi = _sc_gather_sweep(
        user_id.astype(jnp.int32), item_id.astype(jnp.int32), utab3, itab3)
    return _tc_mlp(
        xu, xi, W1, b1.reshape(1, -1), W2, b2.reshape(1, -1),
        W3, b3.reshape(1, -1), batch, blk=2048)


# speedup vs baseline: 17.6960x; 17.6960x over previous
import functools

import jax
import jax.numpy as jnp
from jax import lax
from jax.experimental import pallas as pl
from jax.experimental.pallas import tpu as pltpu
from jax.experimental.pallas import tpu_sc as plsc

_NC = 2
_NS = 16
_NW = _NC * _NS
_GRP = 16
_DEPTH = 4
_FLUSH = 64


def _sc_gather_concat(uidx, iidx, utab3, itab3):
    batch = uidx.shape[0]
    n_tr, n_sub, _ = utab3.shape
    hidden = n_tr * n_sub
    b_per_w = batch // _NW
    n_grp = b_per_w // _GRP
    mesh = plsc.VectorSubcoreMesh(core_axis_name="c", subcore_axis_name="s")

    block = pltpu.VMEM((n_tr, n_sub, 128), jnp.float32)

    @functools.partial(
        pl.kernel,
        mesh=mesh,
        compiler_params=pltpu.CompilerParams(needs_layout_passes=False),
        out_type=jax.ShapeDtypeStruct((batch, 2 * hidden), jnp.float32),
        scratch_types=(
            pltpu.VMEM((b_per_w,), jnp.int32),
            pltpu.VMEM((b_per_w,), jnp.int32),
            [block] * _DEPTH,
            [block] * _DEPTH,
            pltpu.VMEM((_FLUSH, 2 * hidden), jnp.float32),
            [pltpu.SemaphoreType.DMA] * _DEPTH,
        ),
    )
    def gather_kernel(uidx_hbm, iidx_hbm, utab_hbm, itab_hbm, out_hbm,
                      idx_u, idx_i, bus, bis, stage, sems):
        wid = lax.axis_index("s") * _NC + lax.axis_index("c")
        base = wid * b_per_w
        pltpu.sync_copy(uidx_hbm.at[pl.ds(base, b_per_w)], idx_u)
        pltpu.sync_copy(iidx_hbm.at[pl.ds(base, b_per_w)], idx_i)

        lanes = lax.iota(jnp.int32, 16)
        trq = [(lanes + 16 * q) >> 3 for q in range(hidden // 16)]
        sq = [(lanes + 16 * q) & 7 for q in range(hidden // 16)]

        def fire(uid, iid, d):
            uoff = pl.multiple_of((uid >> 7) * 128, 128)
            ioff = pl.multiple_of((iid >> 7) * 128, 128)
            half = n_tr // 2
            for p in range(2):
                tr = pl.ds(p * half, half)
                pltpu.async_copy(
                    utab_hbm.at[tr, :, pl.ds(uoff, 128)],
                    bus[d].at[tr], sems[d])
                pltpu.async_copy(
                    itab_hbm.at[tr, :, pl.ds(ioff, 128)],
                    bis[d].at[tr], sems[d])

        def wait_pair(d):
            pltpu.make_async_copy(utab_hbm.at[:, :, pl.ds(0, 128)], bus[d],
                                  sems[d]).wait()
            pltpu.make_async_copy(itab_hbm.at[:, :, pl.ds(0, 128)], bis[d],
                                  sems[d]).wait()

        def extract(uid, iid, r, d):
            lu = lanes * 0 + (uid & 127)
            li = lanes * 0 + (iid & 127)
            for q in range(hidden // 16):
                vu = plsc.load_gather(bus[d], [trq[q], sq[q], lu])
                vi = plsc.load_gather(bis[d], [trq[q], sq[q], li])
                stage[r, pl.ds(16 * q, 16)] = vu
                stage[r, pl.ds(hidden + 16 * q, 16)] = vi

        def load_ids(g):
            vu = idx_u[pl.ds(g * _GRP, _GRP)]
            vi = idx_i[pl.ds(g * _GRP, _GRP)]
            return ([vu[l] for l in range(_GRP)],
                    [vi[l] for l in range(_GRP)])

        uids0, iids0 = load_ids(0)
        for l in range(_DEPTH):
            fire(uids0[l], iids0[l], l)

        def group_body(g, carry):
            uids, iids = load_ids(g)
            gnext = jnp.minimum(g + 1, n_grp - 1)
            uids1, iids1 = load_ids(gnext)
            for l in range(_GRP):
                d = l % _DEPTH
                wait_pair(d)
                extract(uids[l], iids[l], (g * _GRP + l) & (_FLUSH - 1), d)
                if l + _DEPTH < _GRP:
                    fire(uids[l + _DEPTH], iids[l + _DEPTH], d)
                else:
                    ln = l + _DEPTH - _GRP

                    @pl.when(g < n_grp - 1)
                    def _():
                        fire(uids1[ln], iids1[ln], d)

            @pl.when((g & 3) == 3)
            def _():
                row0 = pl.multiple_of(base + ((g >> 2) << 6), _FLUSH)
                pltpu.sync_copy(stage, out_hbm.at[pl.ds(row0, _FLUSH), :])

            return carry

        lax.fori_loop(0, n_grp, group_body, 0)

    return gather_kernel(uidx, iidx, utab3, itab3)


def _mlp_body(x_ref, w1_ref, b1_ref, w2_ref, b2_ref, w3_ref, b3_ref, o_ref):
    h = jnp.dot(x_ref[...], w1_ref[...], preferred_element_type=jnp.float32)
    h = jnp.maximum(h + b1_ref[...], 0.0)
    h = jnp.maximum(
        jnp.dot(h, w2_ref[...], preferred_element_type=jnp.float32)
        + b2_ref[...], 0.0)
    o_ref[...] = (jnp.dot(h, w3_ref[...], preferred_element_type=jnp.float32)
                  + b3_ref[...])


def _tc_mlp(x, w1, b1r, w2, b2r, w3, b3r, blk):
    batch, d_in = x.shape
    d1 = w1.shape[1]
    d2 = w2.shape[1]
    d3 = w3.shape[1]
    grid = (batch // blk,)
    fixed = lambda b: (0, 0)
    return pl.pallas_call(
        _mlp_body,
        grid=grid,
        in_specs=[
            pl.BlockSpec((blk, d_in), lambda b: (b, 0)),
            pl.BlockSpec((d_in, d1), fixed),
            pl.BlockSpec((1, d1), fixed),
            pl.BlockSpec((d1, d2), fixed),
            pl.BlockSpec((1, d2), fixed),
            pl.BlockSpec((d2, d3), fixed),
            pl.BlockSpec((1, d3), fixed),
        ],
        out_specs=pl.BlockSpec((blk, d3), lambda b: (b, 0)),
        out_shape=jax.ShapeDtypeStruct((batch, d3), jnp.float32),
    )(x, w1, b1r, w2, b2r, w3, b3r)


def kernel(user_id, item_id, user_table, item_table, W1, b1, W2, b2, W3, b3):
    n_ids, hidden = user_table.shape
    utab3 = user_table.T.reshape(8, hidden // 8, n_ids)
    itab3 = item_table.T.reshape(8, hidden // 8, n_ids)
    x = _sc_gather_concat(
        user_id.astype(jnp.int32), item_id.astype(jnp.int32), utab3, itab3)
    return _tc_mlp(
        x, W1, b1.reshape(1, -1), W2, b2.reshape(1, -1),
        W3, b3.reshape(1, -1), blk=2048)
